# speedup vs baseline: 2.1897x; 2.1897x over previous
import functools

import jax
import jax.numpy as jnp
from jax import lax
from jax.experimental import pallas as pl
from jax.experimental.pallas import tpu as pltpu
from jax.experimental.pallas import tpu_sc as plsc

_RATIO = 0.75
_T, _B, _C = 576, 64, 768
_REMAIN_T = int(_T * (1 - _RATIO))

_NC, _NS = 2, 16
_NW = _NC * _NS
_ROWS = _REMAIN_T * _B
_ROWS_PER_W = _ROWS // _NW
_N_CHUNKS = 4
_CHUNK = _ROWS_PER_W // _N_CHUNKS


@functools.partial(
    pl.kernel,
    mesh=plsc.VectorSubcoreMesh(core_axis_name="c", subcore_axis_name="s"),
    out_type=jax.ShapeDtypeStruct((_ROWS, _C), jnp.float32),
    scratch_types=[
        pltpu.VMEM((_N_CHUNKS, _CHUNK), jnp.int32),
        pltpu.VMEM((_CHUNK, _C), jnp.float32),
        pltpu.SemaphoreType.DMA,
    ],
)
def _gather_rows(table_hbm, idx_hbm, out_hbm, idx_v, rows_v, sem):
    wid = lax.axis_index("s") * _NC + lax.axis_index("c")
    pltpu.sync_copy(idx_hbm.at[wid], idx_v)
    base = wid * _ROWS_PER_W
    for j in range(_N_CHUNKS):
        pltpu.async_copy(table_hbm.at[idx_v.at[j]], rows_v, sem).wait()
        pltpu.sync_copy(rows_v, out_hbm.at[pl.ds(base + j * _CHUNK, _CHUNK)])


def kernel(patches):
    noise = jax.random.uniform(jax.random.key(42), (_T, _B), dtype=jnp.float32)
    fwd = jnp.argsort(noise, axis=0)
    bwd = jnp.argsort(fwd, axis=0)
    flat_idx = (
        fwd[:_REMAIN_T].astype(jnp.int32) * _B
        + jnp.arange(_B, dtype=jnp.int32)[None, :]
    ).reshape(_NW, _N_CHUNKS, _CHUNK)
    table = patches.reshape(_T * _B, _C)
    out_flat = _gather_rows(table, flat_idx)
    return (out_flat.reshape(_REMAIN_T, _B, _C), fwd, bwd)


# speedup vs baseline: 2.2723x; 1.0377x over previous
import functools

import jax
import jax.numpy as jnp
from jax import lax
from jax.experimental import pallas as pl
from jax.experimental.pallas import tpu as pltpu
from jax.experimental.pallas import tpu_sc as plsc

_RATIO = 0.75
_T, _B, _C = 576, 64, 768
_REMAIN_T = int(_T * (1 - _RATIO))

_NC, _NS = 2, 16
_NW = _NC * _NS
_ROWS = _REMAIN_T * _B
_ROWS_PER_W = _ROWS // _NW
_N_CHUNKS = 6
_CHUNK = _ROWS_PER_W // _N_CHUNKS
_NBUF = 3


@functools.partial(
    pl.kernel,
    mesh=plsc.VectorSubcoreMesh(core_axis_name="c", subcore_axis_name="s"),
    out_type=jax.ShapeDtypeStruct((_ROWS, _C), jnp.float32),
    scratch_types=[
        pltpu.VMEM((_N_CHUNKS, _CHUNK), jnp.int32),
        pltpu.VMEM((_CHUNK, _C), jnp.float32),
        pltpu.VMEM((_CHUNK, _C), jnp.float32),
        pltpu.VMEM((_CHUNK, _C), jnp.float32),
        pltpu.SemaphoreType.DMA((_NBUF,)),
        pltpu.SemaphoreType.DMA((_NBUF,)),
    ],
)
def _gather_rows(table_hbm, idx_hbm, out_hbm, idx_v, r0, r1, r2, gsem, ssem):
    wid = lax.axis_index("s") * _NC + lax.axis_index("c")
    pltpu.sync_copy(idx_hbm.at[wid], idx_v)
    base = wid * _ROWS_PER_W
    bufs = (r0, r1, r2)
    gcp = [None] * _N_CHUNKS
    scp = [None] * _N_CHUNKS
    for j in range(_NBUF):
        gcp[j] = pltpu.async_copy(table_hbm.at[idx_v.at[j]], bufs[j], gsem.at[j])
    for j in range(_N_CHUNKS):
        b = j % _NBUF
        gcp[j].wait()
        scp[j] = pltpu.async_copy(
            bufs[b], out_hbm.at[pl.ds(base + j * _CHUNK, _CHUNK)], ssem.at[b]
        )
        nxt = j + _NBUF
        if nxt < _N_CHUNKS:
            scp[j].wait()
            gcp[nxt] = pltpu.async_copy(
                table_hbm.at[idx_v.at[nxt]], bufs[b], gsem.at[b]
            )
    for j in range(_N_CHUNKS - _NBUF, _N_CHUNKS):
        scp[j].wait()


def kernel(patches):
    noise = jax.random.uniform(jax.random.key(42), (_T, _B), dtype=jnp.float32)
    fwd = jnp.argsort(noise, axis=0)
    bwd = jnp.argsort(fwd, axis=0)
    flat_idx = (
        fwd[:_REMAIN_T].astype(jnp.int32) * _B
        + jnp.arange(_B, dtype=jnp.int32)[None, :]
    ).reshape(_NW, _N_CHUNKS, _CHUNK)
    table = patches.reshape(_T * _B, _C)
    out_flat = _gather_rows(table, flat_idx)
    return (out_flat.reshape(_REMAIN_T, _B, _C), fwd, bwd)
